# baseline (speedup 1.0000x reference)
import functools

import jax
import jax.numpy as jnp
from jax import lax
from jax.experimental import pallas as pl
from jax.experimental.pallas import tpu as pltpu
from jax.experimental.pallas import tpu_sc as plsc

NC = 2
NS = 16
NW = NC * NS
NBUF = 4
EPS = 1e-12


def _make_sc_gather(n_pairs, vocab, embed):
    mesh = plsc.VectorSubcoreMesh(
        core_axis_name="c", subcore_axis_name="s", num_cores=NC, num_subcores=NS
    )
    c_pairs = 200
    n_iter = n_pairs // (NW * c_pairs)
    ew = 2 * embed

    @functools.partial(
        pl.kernel,
        mesh=mesh,
        out_type=jax.ShapeDtypeStruct((n_pairs, ew), jnp.float32),
        compiler_params=pltpu.CompilerParams(
            needs_layout_passes=False, use_tc_tiling_on_sc=False
        ),
        scratch_types=[
            [pltpu.VMEM((c_pairs,), jnp.int32) for _ in range(NBUF)],
            [pltpu.VMEM((c_pairs,), jnp.int32) for _ in range(NBUF)],
            [pltpu.VMEM((c_pairs, embed), jnp.float32) for _ in range(NBUF)],
            [pltpu.VMEM((c_pairs, embed), jnp.float32) for _ in range(NBUF)],
            [pltpu.SemaphoreType.DMA for _ in range(NBUF)],
            [pltpu.SemaphoreType.DMA for _ in range(NBUF)],
            [pltpu.SemaphoreType.DMA for _ in range(NBUF)],
        ],
    )
    def body(ids_hbm, table_hbm, out_hbm, idxes, idxos, bufes, bufos,
             sxs, sis, sos):
        w = lax.axis_index("s") * NC + lax.axis_index("c")

        def u0(i):
            return (w * n_iter + i) * c_pairs

        def start_idx(i, b):
            pltpu.async_copy(ids_hbm.at[0, pl.ds(u0(i), c_pairs)], idxes[b],
                             sxs[b])
            pltpu.async_copy(ids_hbm.at[1, pl.ds(u0(i), c_pairs)], idxos[b],
                             sxs[b])

        def wait_idx(i, b):
            pltpu.make_async_copy(ids_hbm.at[0, pl.ds(u0(i), c_pairs)],
                                  idxes[b], sxs[b]).wait()
            pltpu.make_async_copy(ids_hbm.at[1, pl.ds(u0(i), c_pairs)],
                                  idxos[b], sxs[b]).wait()

        def start_in(b):
            pltpu.async_copy(table_hbm.at[idxes[b]], bufes[b], sis[b])
            pltpu.async_copy(table_hbm.at[idxos[b]], bufos[b], sis[b])

        def wait_in(b):
            pltpu.make_async_copy(table_hbm.at[idxes[b]], bufes[b],
                                  sis[b]).wait()
            pltpu.make_async_copy(table_hbm.at[idxos[b]], bufos[b],
                                  sis[b]).wait()

        def out_halves(i, b):
            rows = out_hbm.at[pl.ds(u0(i), c_pairs)]
            return ((bufes[b], rows.at[:, pl.ds(0, embed)]),
                    (bufos[b], rows.at[:, pl.ds(embed, embed)]))

        def start_out(i, b):
            for src, dst in out_halves(i, b):
                pltpu.async_copy(src, dst, sos[b])

        def wait_out(i, b):
            for src, dst in out_halves(i, b):
                pltpu.make_async_copy(src, dst, sos[b]).wait()

        start_idx(0, 0)
        start_idx(1, 1)
        start_idx(2, 2)
        wait_idx(0, 0)
        start_in(0)
        wait_idx(1, 1)
        start_in(1)

        def step(i, b):
            wait_in(b)

            b3 = (b + 3) % NBUF

            @pl.when(i + 3 < n_iter)
            def _():
                start_idx(i + 3, b3)

            start_out(i, b)

            j = i + 2
            b2 = (b + 2) % NBUF

            @pl.when(j < n_iter)
            def _():
                @pl.when(j >= NBUF)
                def _():
                    wait_out(j - NBUF, b2)

                wait_idx(j, b2)
                start_in(b2)

        def outer(o, _):
            for b in range(NBUF):
                step(o * NBUF + b, b)
            return 0

        lax.fori_loop(0, n_iter // NBUF, outer, 0)
        for b in range(NBUF):
            wait_out(n_iter - NBUF + b, b)

    return body


def _tc_ln_body(gref, gm, bt, oref):
    x = gref[...]
    e = gm.shape[1] // 2

    def stats(v):
        m = jnp.mean(v, axis=1, keepdims=True)
        var = jnp.maximum(jnp.mean(v * v, axis=1, keepdims=True) - m * m, 0.0)
        return m, lax.rsqrt(var + EPS)

    ml, rl = stats(x[:, :e])
    mr, rr = stats(x[:, e:])
    rows = x.shape[0]
    m = jnp.concatenate([jnp.broadcast_to(ml, (rows, e)),
                         jnp.broadcast_to(mr, (rows, e))], axis=1)
    r = jnp.concatenate([jnp.broadcast_to(rl, (rows, e)),
                         jnp.broadcast_to(rr, (rows, e))], axis=1)
    oref[...] = (x - m) * r * gm[...] + bt[...]


def _make_tc_ln(n_pairs, ew, block_rows):
    grid = n_pairs // block_rows
    return pl.pallas_call(
        _tc_ln_body,
        grid=(grid,),
        in_specs=[
            pl.BlockSpec((block_rows, ew), lambda i: (i, 0)),
            pl.BlockSpec((1, ew), lambda i: (0, 0)),
            pl.BlockSpec((1, ew), lambda i: (0, 0)),
        ],
        out_specs=pl.BlockSpec((block_rows, ew), lambda i: (i, 0)),
        out_shape=jax.ShapeDtypeStruct((n_pairs, ew), jnp.float32),
    )


def kernel(input_ids, table, gamma, beta):
    n_batch, seq = input_ids.shape
    vocab, embed = table.shape
    n_rows = n_batch * seq
    n_pairs = n_rows // 2
    assert n_pairs % (NW * 200) == 0 and embed % 16 == 0

    ids_eo = input_ids.astype(jnp.int32).reshape(n_pairs, 2).T
    g = _make_sc_gather(n_pairs, vocab, embed)(ids_eo, table)

    gamma2 = jnp.tile(gamma, 2).reshape(1, 2 * embed)
    beta2 = jnp.tile(beta, 2).reshape(1, 2 * embed)
    y = _make_tc_ln(n_pairs, 2 * embed, 3200)(g, gamma2, beta2)
    return y.reshape(n_batch, seq, embed)


# speedup vs baseline: 1.2866x; 1.2866x over previous
import functools

import jax
import jax.numpy as jnp
from jax import lax
from jax.experimental import pallas as pl
from jax.experimental.pallas import tpu as pltpu
from jax.experimental.pallas import tpu_sc as plsc

NC = 2
NS = 16
NW = NC * NS
LANES = 16
NBUF = 4
BPC = 2
EPS = 1e-12


def _fast_rsqrt(x):
    i = plsc.bitcast(x, jnp.int32)
    i = jnp.int32(0x5F3759DF) - lax.shift_right_logical(i, 1)
    y = plsc.bitcast(i, jnp.float32)
    for _ in range(1):
        y = y * (1.5 - 0.5 * x * y * y)
    return y


def _make_sc_kernel(n_batch, seq, embed):
    mesh = plsc.VectorSubcoreMesh(
        core_axis_name="c", subcore_axis_name="s", num_cores=NC, num_subcores=NS
    )
    c_rows = BPC * seq
    n_iter = n_batch // (NW * BPC)

    @functools.partial(
        pl.kernel,
        mesh=mesh,
        out_type=jax.ShapeDtypeStruct((n_batch, seq, embed), jnp.float32),
        compiler_params=pltpu.CompilerParams(
            needs_layout_passes=False, use_tc_tiling_on_sc=False
        ),
        scratch_types=[
            [pltpu.VMEM((c_rows,), jnp.int32) for _ in range(NBUF)],
            [pltpu.VMEM((c_rows, embed), jnp.float32) for _ in range(NBUF)],
            pltpu.VMEM((embed,), jnp.float32),
            pltpu.VMEM((embed,), jnp.float32),
            [pltpu.SemaphoreType.DMA for _ in range(NBUF)],
            [pltpu.SemaphoreType.DMA for _ in range(NBUF)],
            [pltpu.SemaphoreType.DMA for _ in range(NBUF)],
        ],
    )
    def body(ids_hbm, table_hbm, gamma_hbm, beta_hbm, out_hbm,
             idxbs, bufs, gv, bv, sxs, sis, sos):
        w = lax.axis_index("s") * NC + lax.axis_index("c")
        pltpu.sync_copy(gamma_hbm, gv)
        pltpu.sync_copy(beta_hbm, bv)

        lane = lax.iota(jnp.int32, LANES)
        inv_e = jnp.float32(1.0 / embed)

        perms = [lane ^ (1 << t) for t in range(4)]

        def xlsum(v):
            for p in perms:
                v = v + jnp.take_along_axis(
                    v, p, axis=0, mode="promise_in_bounds"
                )
            return v

        nq = embed // LANES

        def compute(buf):
            gvecs = [gv[pl.ds(k * LANES, LANES)] for k in range(nq)]
            bvecs = [bv[pl.ds(k * LANES, LANES)] for k in range(nq)]

            @plsc.parallel_loop(0, c_rows, unroll=4)
            def ln_row(r):
                xs = [buf[r, pl.ds(k * LANES, LANES)] for k in range(nq)]
                s = xs[0] + xs[1] + xs[2] + xs[3]
                sq = (xs[0] * xs[0] + xs[1] * xs[1]
                      + xs[2] * xs[2] + xs[3] * xs[3])
                mean = xlsum(s) * inv_e
                var = jnp.maximum(xlsum(sq) * inv_e - mean * mean, 0.0)
                rstd = _fast_rsqrt(var + EPS)
                m2 = mean * rstd
                for k in range(nq):
                    y = (xs[k] * rstd - m2) * gvecs[k] + bvecs[k]
                    buf[r, pl.ds(k * LANES, LANES)] = y

        def batch0(i):
            return (w * n_iter + i) * BPC

        def start_idx(i, b):
            pltpu.async_copy(ids_hbm.at[pl.ds(batch0(i) * seq, c_rows)],
                             idxbs[b], sxs[b])

        def wait_idx(i, b):
            pltpu.make_async_copy(
                ids_hbm.at[pl.ds(batch0(i) * seq, c_rows)], idxbs[b], sxs[b]
            ).wait()

        def start_in(b):
            pltpu.async_copy(table_hbm.at[idxbs[b]], bufs[b], sis[b])

        def wait_in(b):
            pltpu.make_async_copy(table_hbm.at[idxbs[b]], bufs[b], sis[b]).wait()

        def start_out(i, b):
            for j in range(BPC):
                pltpu.async_copy(bufs[b].at[pl.ds(j * seq, seq)],
                                 out_hbm.at[batch0(i) + j], sos[b])

        def wait_out(i, b):
            for j in range(BPC):
                pltpu.make_async_copy(
                    bufs[b].at[pl.ds(j * seq, seq)],
                    out_hbm.at[batch0(i) + j], sos[b]
                ).wait()

        start_idx(0, 0)
        start_idx(1, 1)
        start_idx(2, 2)
        wait_idx(0, 0)
        start_in(0)
        wait_idx(1, 1)
        start_in(1)

        def step(i, b):
            wait_in(b)

            b3 = (b + 3) % NBUF

            @pl.when(i + 3 < n_iter)
            def _():
                start_idx(i + 3, b3)

            compute(bufs[b])
            start_out(i, b)

            j = i + 2
            b2 = (b + 2) % NBUF

            @pl.when(j < n_iter)
            def _():
                @pl.when(j >= NBUF)
                def _():
                    wait_out(j - NBUF, b2)

                wait_idx(j, b2)
                start_in(b2)

        def outer(o, _):
            for b in range(NBUF):
                step(o * NBUF + b, b)
            return 0

        lax.fori_loop(0, n_iter // NBUF, outer, 0)
        for b in range(NBUF):
            wait_out(n_iter - NBUF + b, b)

    return body


def kernel(input_ids, table, gamma, beta):
    n_batch, seq = input_ids.shape
    vocab, embed = table.shape
    assert n_batch % (NW * BPC) == 0 and embed % LANES == 0

    sc = _make_sc_kernel(n_batch, seq, embed)
    ids = input_ids.astype(jnp.int32).reshape(-1)
    return sc(ids, table, gamma, beta)
